# baseline (speedup 1.0000x reference)
import jax
import jax.numpy as jnp
from jax import lax
from jax.experimental import pallas as pl
from jax.experimental.pallas import tpu as pltpu
from jax.experimental.pallas import tpu_sc as plsc

N = 10000
E = 320000
D = 128
CK = 128
NCHUNKS = E // CK
CK1 = 80
NCHUNKS1 = E // CK1
NC, NS = 2, 16
NW = NC * NS
N_PAD = 10112
ROWS_PER_SUB = N_PAD // NS
_SLICE_CHUNKS = [(0, 80), (80, 80), (160, 80), (240, 80), (320, 80),
                 (400, 80), (480, 80), (560, 72)]
BE = 512
BN = 2000


def _zero_rows(buf, nrows):
    def body(r, _):
        for j in range(D // 16):
            buf[r, pl.ds(j * 16, 16)] = jnp.zeros((16,), jnp.float32)
        return 0
    lax.fori_loop(0, nrows, body, 0)


def _zero_acc_slice(buf, acc, s):
    _zero_rows(buf, 80)
    for off, ln in _SLICE_CHUNKS:
        pltpu.sync_copy(buf.at[pl.ds(0, ln)],
                        acc.at[pl.ds(s * ROWS_PER_SUB + off, ln)])


def _readout_acc_slice(acc, out_hbm, c, s):
    for off, ln in _SLICE_CHUNKS:
        sl = pl.ds(s * ROWS_PER_SUB + off, ln)
        pltpu.sync_copy(acc.at[sl], out_hbm.at[c, sl])


def _k1_body(h_hbm, src_hbm, dst_hbm,
             prod_hbm, sd_part_hbm, norms_hbm,
             idx_s0, idx_d0, idx_s1, idx_d1,
             hs0, hd0, hs1, hd1, nrm_v, acc,
             sem_s0, sem_d0, sem_s1, sem_d1):
    c = lax.axis_index("c")
    s = lax.axis_index("s")
    wid = s * NC + c
    idx_s = (idx_s0, idx_s1)
    idx_d = (idx_d0, idx_d1)
    hs = (hs0, hs1)
    hd = (hd0, hd1)
    sem_s = (sem_s0, sem_s1)
    sem_d = (sem_d0, sem_d1)

    _zero_acc_slice(hs0, acc, s)
    nrm_v[0, :] = jnp.zeros((16,), jnp.float32)
    nrm_v[1, :] = jnp.zeros((16,), jnp.float32)
    plsc.subcore_barrier()

    nw = NCHUNKS1 // NW

    def start_gather(t, b):
        base = (wid + t * NW) * CK1
        pltpu.sync_copy(src_hbm.at[pl.ds(base, CK1)], idx_s[b])
        pltpu.sync_copy(dst_hbm.at[pl.ds(base, CK1)], idx_d[b])
        pltpu.async_copy(h_hbm.at[idx_s[b]], hs[b], sem_s[b])
        pltpu.async_copy(h_hbm.at[idx_d[b]], hd[b], sem_d[b])

    start_gather(0, 0)
    start_gather(1, 1)

    def process(t, b, start_next):
        base = (wid + t * NW) * CK1
        pltpu.make_async_copy(h_hbm.at[idx_s[b]], hs[b], sem_s[b]).wait()
        pltpu.make_async_copy(h_hbm.at[idx_d[b]], hd[b], sem_d[b]).wait()

        def row_body(r, rc):
            rns, rnd = rc
            for j in range(D // 16):
                ds = pl.ds(j * 16, 16)
                a = hs[b][r, ds]
                bb = hd[b][r, ds]
                hd[b][r, ds] = a * bb
                hs[b][r, ds] = a - bb
                rns = rns + a * a
                rnd = rnd + bb * bb
            return rns, rnd

        z16 = jnp.zeros((16,), jnp.float32)
        rns, rnd = lax.fori_loop(0, CK1, row_body, (z16, z16))
        nrm_v[0, :] = nrm_v[0, :] + rns
        nrm_v[1, :] = nrm_v[1, :] + rnd
        pltpu.sync_copy(hd[b], prod_hbm.at[pl.ds(base, CK1)])
        pltpu.sync_copy(hs[b], acc.at[idx_d[b]], add=True)
        if start_next:
            start_gather(t + 2, b)

    def pair_body(i2, _):
        for b in range(2):
            t = i2 * 2 + b

            @pl.when(t + 2 < nw)
            def _(t=t, b=b):
                process(t, b, True)

            @pl.when(t + 2 >= nw)
            def _(t=t, b=b):
                process(t, b, False)
        return 0

    lax.fori_loop(0, nw // 2, pair_body, 0)
    process(nw - 1, (nw - 1) % 2, False)

    pltpu.sync_copy(nrm_v, norms_hbm.at[:, wid])
    plsc.subcore_barrier()
    _readout_acc_slice(acc, sd_part_hbm, c, s)


def _k3_body(sd_hbm, src_hbm, dst_hbm, score_hbm,
             hdp_hbm,
             idx_s0, idx_d0, idx_s1, idx_d1,
             sc0, sc1, buf0, buf1, acc,
             sem0, sem1):
    c = lax.axis_index("c")
    s = lax.axis_index("s")
    wid = s * NC + c
    idx_s = (idx_s0, idx_s1)
    idx_d = (idx_d0, idx_d1)
    sc = (sc0, sc1)
    buf = (buf0, buf1)
    sem = (sem0, sem1)

    _zero_acc_slice(buf0, acc, s)
    plsc.subcore_barrier()

    nw = (NCHUNKS - wid + NW - 1) // NW

    def start_gather(t, b):
        base = (wid + t * NW) * CK
        pltpu.sync_copy(src_hbm.at[pl.ds(base, CK)], idx_s[b])
        pltpu.sync_copy(dst_hbm.at[pl.ds(base, CK)], idx_d[b])
        pltpu.sync_copy(score_hbm.at[pl.ds(base, CK)], sc[b])
        pltpu.async_copy(sd_hbm.at[idx_s[b]], buf[b], sem[b])

    for b in range(2):
        @pl.when(b < nw)
        def _(b=b):
            start_gather(b, b)

    def process(t, b):
        pltpu.make_async_copy(sd_hbm.at[idx_s[b]], buf[b], sem[b]).wait()

        def row_body(r, _rc):
            grp = sc[b][pl.ds((r // 16) * 16, 16)]
            lane = jnp.full((16,), r % 16, jnp.int32)
            sval = lax.gather(
                grp, lane[:, None],
                lax.GatherDimensionNumbers(offset_dims=(),
                                           collapsed_slice_dims=(0,),
                                           start_index_map=(0,)),
                (1,), mode=lax.GatherScatterMode.PROMISE_IN_BOUNDS)
            for j in range(D // 16):
                ds = pl.ds(j * 16, 16)
                buf[b][r, ds] = buf[b][r, ds] * sval
            return 0

        lax.fori_loop(0, CK, row_body, 0)
        pltpu.sync_copy(buf[b], acc.at[idx_d[b]], add=True)

        @pl.when(t + 2 < nw)
        def _():
            start_gather(t + 2, b)

    def pair_body(i2, _):
        for b in range(2):
            t = i2 * 2 + b

            @pl.when(t < nw)
            def _(t=t, b=b):
                process(t, b)
        return 0

    lax.fori_loop(0, (nw + 1) // 2, pair_body, 0)

    plsc.subcore_barrier()
    _readout_acc_slice(acc, hdp_hbm, c, s)


def _score_body(norms_ref, prod_ref, p_ref, out_ref):
    nsq = jnp.sum(norms_ref[...], axis=1)
    scale = jnp.sqrt(nsq[0]) * jnp.sqrt(nsq[1]) + 1e-06
    x = prod_ref[...]
    xh = x.astype(jnp.bfloat16)
    xl = (x - xh.astype(jnp.float32)).astype(jnp.bfloat16)
    p = p_ref[...]
    ph = p.astype(jnp.bfloat16)
    pl_ = (p - ph.astype(jnp.float32)).astype(jnp.bfloat16)
    t = (jnp.dot(xh, ph, preferred_element_type=jnp.float32)
         + jnp.dot(xh, pl_, preferred_element_type=jnp.float32)
         + jnp.dot(xl, ph, preferred_element_type=jnp.float32))
    t = jax.nn.relu(t)
    out_ref[...] = jnp.exp(jnp.clip(jnp.sum(t, axis=1) / scale, -5.0, 5.0))


def _combine_body(a_ref, out_ref):
    out_ref[...] = a_ref[0] + a_ref[1]


def _ffn_body(hp_ref, w_ref, b_ref, out_ref):
    x = hp_ref[0] + hp_ref[1]
    y = lax.dot_general(x, w_ref[...], (((1,), (1,)), ((), ())),
                        preferred_element_type=jnp.float32)
    out_ref[...] = jax.nn.relu(y + b_ref[...])


_sc_mesh = plsc.VectorSubcoreMesh(core_axis_name="c", subcore_axis_name="s")

_k1 = pl.kernel(
    _k1_body,
    out_type=[
        jax.ShapeDtypeStruct((E, D), jnp.float32),
        jax.ShapeDtypeStruct((NC, N_PAD, D), jnp.float32),
        jax.ShapeDtypeStruct((2, NW, 16), jnp.float32),
    ],
    mesh=_sc_mesh,
    scratch_types=[
        pltpu.VMEM((CK1,), jnp.int32),
        pltpu.VMEM((CK1,), jnp.int32),
        pltpu.VMEM((CK1,), jnp.int32),
        pltpu.VMEM((CK1,), jnp.int32),
        pltpu.VMEM((CK1, D), jnp.float32),
        pltpu.VMEM((CK1, D), jnp.float32),
        pltpu.VMEM((CK1, D), jnp.float32),
        pltpu.VMEM((CK1, D), jnp.float32),
        pltpu.VMEM((2, 16), jnp.float32),
        pltpu.VMEM_SHARED((N_PAD, D), jnp.float32),
        pltpu.SemaphoreType.DMA,
        pltpu.SemaphoreType.DMA,
        pltpu.SemaphoreType.DMA,
        pltpu.SemaphoreType.DMA,
    ],
    name="k1_gather_prod_segsum",
)

_k3 = pl.kernel(
    _k3_body,
    out_type=jax.ShapeDtypeStruct((NC, N_PAD, D), jnp.float32),
    mesh=_sc_mesh,
    scratch_types=[
        pltpu.VMEM((CK,), jnp.int32),
        pltpu.VMEM((CK,), jnp.int32),
        pltpu.VMEM((CK,), jnp.int32),
        pltpu.VMEM((CK,), jnp.int32),
        pltpu.VMEM((CK,), jnp.float32),
        pltpu.VMEM((CK,), jnp.float32),
        pltpu.VMEM((CK, D), jnp.float32),
        pltpu.VMEM((CK, D), jnp.float32),
        pltpu.VMEM_SHARED((N_PAD, D), jnp.float32),
        pltpu.SemaphoreType.DMA,
        pltpu.SemaphoreType.DMA,
    ],
    name="k3_weighted_segsum",
)


def kernel(h, edge_index, proj_cosim, ffn_w, ffn_b):
    src = edge_index[0]
    dst = edge_index[1]

    prod, sd_part, norms = _k1(h, src, dst)

    score = pl.pallas_call(
        _score_body,
        grid=(E // BE,),
        in_specs=[
            pl.BlockSpec((2, NW * 16), lambda i: (0, 0)),
            pl.BlockSpec((BE, D), lambda i: (i, 0)),
            pl.BlockSpec((D, D), lambda i: (0, 0)),
        ],
        out_specs=pl.BlockSpec((BE,), lambda i: (i,)),
        out_shape=jax.ShapeDtypeStruct((E,), jnp.float32),
    )(norms.reshape(2, NW * 16), prod, proj_cosim)

    sd = pl.pallas_call(
        _combine_body,
        grid=(4,),
        in_specs=[pl.BlockSpec((NC, 2528, D), lambda i: (0, i, 0))],
        out_specs=pl.BlockSpec((2528, D), lambda i: (i, 0)),
        out_shape=jax.ShapeDtypeStruct((N_PAD, D), jnp.float32),
    )(sd_part)

    hd_part = _k3(sd, src, dst, score)

    out = pl.pallas_call(
        _ffn_body,
        grid=(N // BN,),
        in_specs=[
            pl.BlockSpec((NC, BN, D), lambda i: (0, i, 0)),
            pl.BlockSpec((D, D), lambda i: (0, 0)),
            pl.BlockSpec((1, D), lambda i: (0, 0)),
        ],
        out_specs=pl.BlockSpec((BN, D), lambda i: (i, 0)),
        out_shape=jax.ShapeDtypeStruct((N, D), jnp.float32),
    )(hd_part, ffn_w, ffn_b.reshape(1, D))

    return out


# speedup vs baseline: 1.0828x; 1.0828x over previous
import jax
import jax.numpy as jnp
from jax import lax
from jax.experimental import pallas as pl
from jax.experimental.pallas import tpu as pltpu
from jax.experimental.pallas import tpu_sc as plsc

N = 10000
E = 320000
D = 128
CK = 128
NCHUNKS = E // CK
CK1 = 80
NCHUNKS1 = E // CK1
NC, NS = 2, 16
NW = NC * NS
N_PAD = 10112
ROWS_PER_SUB = N_PAD // NS
_SLICE_CHUNKS = [(0, 80), (80, 80), (160, 80), (240, 80), (320, 80),
                 (400, 80), (480, 80), (560, 72)]
BE = 512
BN = 2000


def _zero_rows(buf, nrows):
    def body(r, _):
        for j in range(D // 16):
            buf[r, pl.ds(j * 16, 16)] = jnp.zeros((16,), jnp.float32)
        return 0
    lax.fori_loop(0, nrows, body, 0)


def _zero_acc_slice(buf, acc, s):
    _zero_rows(buf, 80)
    for off, ln in _SLICE_CHUNKS:
        pltpu.sync_copy(buf.at[pl.ds(0, ln)],
                        acc.at[pl.ds(s * ROWS_PER_SUB + off, ln)])


def _readout_acc_slice(acc, out_hbm, c, s):
    for off, ln in _SLICE_CHUNKS:
        sl = pl.ds(s * ROWS_PER_SUB + off, ln)
        pltpu.sync_copy(acc.at[sl], out_hbm.at[c, sl])


def _k1_body(h_hbm, src_hbm, dst_hbm,
             prod_hbm, sd_part_hbm, norms_hbm,
             idx_s0, idx_d0, idx_s1, idx_d1,
             hs0, hd0, hs1, hd1, nrm_v, acc,
             sem_s0, sem_d0, sem_s1, sem_d1):
    c = lax.axis_index("c")
    s = lax.axis_index("s")
    wid = s * NC + c
    idx_s = (idx_s0, idx_s1)
    idx_d = (idx_d0, idx_d1)
    hs = (hs0, hs1)
    hd = (hd0, hd1)
    sem_s = (sem_s0, sem_s1)
    sem_d = (sem_d0, sem_d1)

    _zero_acc_slice(hs0, acc, s)
    nrm_v[0, :] = jnp.zeros((16,), jnp.float32)
    nrm_v[1, :] = jnp.zeros((16,), jnp.float32)
    plsc.subcore_barrier()

    nw = NCHUNKS1 // NW

    def start_gather(t, b):
        base = (wid + t * NW) * CK1
        pltpu.sync_copy(src_hbm.at[pl.ds(base, CK1)], idx_s[b])
        pltpu.sync_copy(dst_hbm.at[pl.ds(base, CK1)], idx_d[b])
        pltpu.async_copy(h_hbm.at[idx_s[b]], hs[b], sem_s[b])
        pltpu.async_copy(h_hbm.at[idx_d[b]], hd[b], sem_d[b])

    start_gather(0, 0)
    start_gather(1, 1)

    def process(t, b, start_next):
        base = (wid + t * NW) * CK1
        pltpu.make_async_copy(h_hbm.at[idx_s[b]], hs[b], sem_s[b]).wait()
        pltpu.make_async_copy(h_hbm.at[idx_d[b]], hd[b], sem_d[b]).wait()

        def row_body(r, rc):
            rns, rnd = rc
            for j in range(D // 16):
                ds = pl.ds(j * 16, 16)
                a = hs[b][r, ds]
                bb = hd[b][r, ds]
                hd[b][r, ds] = a * bb
                hs[b][r, ds] = a - bb
                rns = rns + a * a
                rnd = rnd + bb * bb
            return rns, rnd

        z16 = jnp.zeros((16,), jnp.float32)
        rns, rnd = lax.fori_loop(0, CK1, row_body, (z16, z16))
        nrm_v[0, :] = nrm_v[0, :] + rns
        nrm_v[1, :] = nrm_v[1, :] + rnd
        pltpu.sync_copy(hd[b], prod_hbm.at[pl.ds(base, CK1)])
        pltpu.sync_copy(hs[b], acc.at[idx_d[b]], add=True)
        if start_next:
            start_gather(t + 2, b)

    def pair_body(i2, _):
        for b in range(2):
            t = i2 * 2 + b

            @pl.when(t + 2 < nw)
            def _(t=t, b=b):
                process(t, b, True)

            @pl.when(t + 2 >= nw)
            def _(t=t, b=b):
                process(t, b, False)
        return 0

    lax.fori_loop(0, nw // 2, pair_body, 0)
    process(nw - 1, (nw - 1) % 2, False)

    pltpu.sync_copy(nrm_v, norms_hbm.at[:, wid])
    plsc.subcore_barrier()
    _readout_acc_slice(acc, sd_part_hbm, c, s)


def _k3_body(sd_hbm, src_hbm, dst_hbm, score_hbm,
             hdp_hbm,
             idx_s0, idx_d0, idx_s1, idx_d1,
             sc0, sc1, buf0, buf1, acc,
             sem0, sem1):
    c = lax.axis_index("c")
    s = lax.axis_index("s")
    wid = s * NC + c
    idx_s = (idx_s0, idx_s1)
    idx_d = (idx_d0, idx_d1)
    sc = (sc0, sc1)
    buf = (buf0, buf1)
    sem = (sem0, sem1)

    _zero_acc_slice(buf0, acc, s)
    plsc.subcore_barrier()

    nw = (NCHUNKS - wid + NW - 1) // NW

    def start_gather(t, b):
        base = (wid + t * NW) * CK
        pltpu.sync_copy(src_hbm.at[pl.ds(base, CK)], idx_s[b])
        pltpu.sync_copy(dst_hbm.at[pl.ds(base, CK)], idx_d[b])
        pltpu.sync_copy(score_hbm.at[pl.ds(base, CK)], sc[b])
        pltpu.async_copy(sd_hbm.at[idx_s[b]], buf[b], sem[b])

    for b in range(2):
        @pl.when(b < nw)
        def _(b=b):
            start_gather(b, b)

    def process(t, b):
        pltpu.make_async_copy(sd_hbm.at[idx_s[b]], buf[b], sem[b]).wait()

        def row_body(r, _rc):
            grp = sc[b][pl.ds((r // 16) * 16, 16)]
            lane = jnp.full((16,), r % 16, jnp.int32)
            sval = lax.gather(
                grp, lane[:, None],
                lax.GatherDimensionNumbers(offset_dims=(),
                                           collapsed_slice_dims=(0,),
                                           start_index_map=(0,)),
                (1,), mode=lax.GatherScatterMode.PROMISE_IN_BOUNDS)
            for j in range(D // 16):
                ds = pl.ds(j * 16, 16)
                buf[b][r, ds] = buf[b][r, ds] * sval
            return 0

        lax.fori_loop(0, CK, row_body, 0)
        pltpu.sync_copy(buf[b], acc.at[idx_d[b]], add=True)

        @pl.when(t + 2 < nw)
        def _():
            start_gather(t + 2, b)

    def pair_body(i2, _):
        for b in range(2):
            t = i2 * 2 + b

            @pl.when(t < nw)
            def _(t=t, b=b):
                process(t, b)
        return 0

    lax.fori_loop(0, (nw + 1) // 2, pair_body, 0)

    plsc.subcore_barrier()
    _readout_acc_slice(acc, hdp_hbm, c, s)


def _score_body(norms_ref, prod_ref, p_ref, out_ref):
    nsq = jnp.sum(norms_ref[...], axis=1)
    scale = jnp.sqrt(nsq[0]) * jnp.sqrt(nsq[1]) + 1e-06
    x = prod_ref[...]
    xh = x.astype(jnp.bfloat16)
    xl = (x - xh.astype(jnp.float32)).astype(jnp.bfloat16)
    p = p_ref[...]
    ph = p.astype(jnp.bfloat16)
    pl_ = (p - ph.astype(jnp.float32)).astype(jnp.bfloat16)
    dn = (((0,), (1,)), ((), ()))
    t = (lax.dot_general(ph, xh, dn, preferred_element_type=jnp.float32)
         + lax.dot_general(pl_, xh, dn, preferred_element_type=jnp.float32)
         + lax.dot_general(ph, xl, dn, preferred_element_type=jnp.float32))
    t = jax.nn.relu(t)
    out_ref[...] = jnp.exp(jnp.clip(jnp.sum(t, axis=0) / scale, -5.0, 5.0))


def _combine_body(a_ref, out_ref):
    out_ref[...] = a_ref[0] + a_ref[1]


def _ffn_body(hp_ref, w_ref, b_ref, out_ref):
    x = hp_ref[0] + hp_ref[1]
    y = lax.dot_general(x, w_ref[...], (((1,), (1,)), ((), ())),
                        preferred_element_type=jnp.float32)
    out_ref[...] = jax.nn.relu(y + b_ref[...])


_sc_mesh = plsc.VectorSubcoreMesh(core_axis_name="c", subcore_axis_name="s")

_k1 = pl.kernel(
    _k1_body,
    out_type=[
        jax.ShapeDtypeStruct((E, D), jnp.float32),
        jax.ShapeDtypeStruct((NC, N_PAD, D), jnp.float32),
        jax.ShapeDtypeStruct((2, NW, 16), jnp.float32),
    ],
    mesh=_sc_mesh,
    scratch_types=[
        pltpu.VMEM((CK1,), jnp.int32),
        pltpu.VMEM((CK1,), jnp.int32),
        pltpu.VMEM((CK1,), jnp.int32),
        pltpu.VMEM((CK1,), jnp.int32),
        pltpu.VMEM((CK1, D), jnp.float32),
        pltpu.VMEM((CK1, D), jnp.float32),
        pltpu.VMEM((CK1, D), jnp.float32),
        pltpu.VMEM((CK1, D), jnp.float32),
        pltpu.VMEM((2, 16), jnp.float32),
        pltpu.VMEM_SHARED((N_PAD, D), jnp.float32),
        pltpu.SemaphoreType.DMA,
        pltpu.SemaphoreType.DMA,
        pltpu.SemaphoreType.DMA,
        pltpu.SemaphoreType.DMA,
    ],
    name="k1_gather_prod_segsum",
)

_k3 = pl.kernel(
    _k3_body,
    out_type=jax.ShapeDtypeStruct((NC, N_PAD, D), jnp.float32),
    mesh=_sc_mesh,
    scratch_types=[
        pltpu.VMEM((CK,), jnp.int32),
        pltpu.VMEM((CK,), jnp.int32),
        pltpu.VMEM((CK,), jnp.int32),
        pltpu.VMEM((CK,), jnp.int32),
        pltpu.VMEM((CK,), jnp.float32),
        pltpu.VMEM((CK,), jnp.float32),
        pltpu.VMEM((CK, D), jnp.float32),
        pltpu.VMEM((CK, D), jnp.float32),
        pltpu.VMEM_SHARED((N_PAD, D), jnp.float32),
        pltpu.SemaphoreType.DMA,
        pltpu.SemaphoreType.DMA,
    ],
    name="k3_weighted_segsum",
)


def kernel(h, edge_index, proj_cosim, ffn_w, ffn_b):
    src = edge_index[0]
    dst = edge_index[1]

    prod, sd_part, norms = _k1(h, src, dst)

    score = pl.pallas_call(
        _score_body,
        grid=(E // BE,),
        in_specs=[
            pl.BlockSpec((2, NW * 16), lambda i: (0, 0)),
            pl.BlockSpec((BE, D), lambda i: (i, 0)),
            pl.BlockSpec((D, D), lambda i: (0, 0)),
        ],
        out_specs=pl.BlockSpec((BE,), lambda i: (i,)),
        out_shape=jax.ShapeDtypeStruct((E,), jnp.float32),
    )(norms.reshape(2, NW * 16), prod, proj_cosim)

    sd = pl.pallas_call(
        _combine_body,
        grid=(4,),
        in_specs=[pl.BlockSpec((NC, 2528, D), lambda i: (0, i, 0))],
        out_specs=pl.BlockSpec((2528, D), lambda i: (i, 0)),
        out_shape=jax.ShapeDtypeStruct((N_PAD, D), jnp.float32),
    )(sd_part)

    hd_part = _k3(sd, src, dst, score)

    out = pl.pallas_call(
        _ffn_body,
        grid=(N // BN,),
        in_specs=[
            pl.BlockSpec((NC, BN, D), lambda i: (0, i, 0)),
            pl.BlockSpec((D, D), lambda i: (0, 0)),
            pl.BlockSpec((1, D), lambda i: (0, 0)),
        ],
        out_specs=pl.BlockSpec((BN, D), lambda i: (i, 0)),
        out_shape=jax.ShapeDtypeStruct((N, D), jnp.float32),
    )(hd_part, ffn_w, ffn_b.reshape(1, D))

    return out


# speedup vs baseline: 1.5398x; 1.4220x over previous
import jax
import jax.numpy as jnp
from jax import lax
from jax.experimental import pallas as pl
from jax.experimental.pallas import tpu as pltpu
from jax.experimental.pallas import tpu_sc as plsc

N = 10000
E = 320000
D = 128
CK = 128
NCHUNKS = E // CK
CK1 = 80
NCHUNKS1 = E // CK1
NC, NS = 2, 16
NW = NC * NS
N_PAD = 10112
ROWS_PER_SUB = N_PAD // NS
_SLICE_CHUNKS = [(0, 80), (80, 80), (160, 80), (240, 80), (320, 80),
                 (400, 80), (480, 80), (560, 72)]
E_PAD = 327680
BE = 8192
BN = 2000


def _zero_rows(buf, nrows):
    def body(r, _):
        for j in range(D // 16):
            buf[r, pl.ds(j * 16, 16)] = jnp.zeros((16,), jnp.float32)
        return 0
    lax.fori_loop(0, nrows, body, 0)


def _zero_acc_slice(buf, acc, s):
    _zero_rows(buf, 80)
    for off, ln in _SLICE_CHUNKS:
        pltpu.sync_copy(buf.at[pl.ds(0, ln)],
                        acc.at[pl.ds(s * ROWS_PER_SUB + off, ln)])


def _readout_acc_slice(acc, out_hbm, c, s):
    for off, ln in _SLICE_CHUNKS:
        sl = pl.ds(s * ROWS_PER_SUB + off, ln)
        pltpu.sync_copy(acc.at[sl], out_hbm.at[c, sl])


def _k1_body(h_hbm, src_hbm, dst_hbm,
             prod_hbm, sd_part_hbm, norms_hbm,
             idx_s0, idx_d0, idx_s1, idx_d1,
             hs0, hd0, hs1, hd1, nrm_v, acc,
             sem_s0, sem_d0, sem_s1, sem_d1):
    c = lax.axis_index("c")
    s = lax.axis_index("s")
    wid = s * NC + c
    idx_s = (idx_s0, idx_s1)
    idx_d = (idx_d0, idx_d1)
    hs = (hs0, hs1)
    hd = (hd0, hd1)
    sem_s = (sem_s0, sem_s1)
    sem_d = (sem_d0, sem_d1)

    _zero_acc_slice(hs0, acc, s)
    nrm_v[0, :] = jnp.zeros((16,), jnp.float32)
    nrm_v[1, :] = jnp.zeros((16,), jnp.float32)
    plsc.subcore_barrier()

    nw = NCHUNKS1 // NW

    def start_gather(t, b):
        base = (wid + t * NW) * CK1
        pltpu.sync_copy(src_hbm.at[pl.ds(base, CK1)], idx_s[b])
        pltpu.sync_copy(dst_hbm.at[pl.ds(base, CK1)], idx_d[b])
        pltpu.async_copy(h_hbm.at[idx_s[b]], hs[b], sem_s[b])
        pltpu.async_copy(h_hbm.at[idx_d[b]], hd[b], sem_d[b])

    start_gather(0, 0)
    start_gather(1, 1)

    def process(t, b, start_next):
        base = (wid + t * NW) * CK1
        pltpu.make_async_copy(h_hbm.at[idx_s[b]], hs[b], sem_s[b]).wait()
        pltpu.make_async_copy(h_hbm.at[idx_d[b]], hd[b], sem_d[b]).wait()

        def row_body(r, rc):
            rns, rnd = rc
            for j in range(D // 16):
                ds = pl.ds(j * 16, 16)
                a = hs[b][r, ds]
                bb = hd[b][r, ds]
                hd[b][r, ds] = a * bb
                hs[b][r, ds] = a - bb
                rns = rns + a * a
                rnd = rnd + bb * bb
            return rns, rnd

        z16 = jnp.zeros((16,), jnp.float32)
        rns, rnd = lax.fori_loop(0, CK1, row_body, (z16, z16))
        nrm_v[0, :] = nrm_v[0, :] + rns
        nrm_v[1, :] = nrm_v[1, :] + rnd
        pltpu.sync_copy(hd[b], prod_hbm.at[pl.ds(base, CK1)])
        pltpu.sync_copy(hs[b], acc.at[idx_d[b]], add=True)
        if start_next:
            start_gather(t + 2, b)

    def pair_body(i2, _):
        for b in range(2):
            t = i2 * 2 + b

            @pl.when(t + 2 < nw)
            def _(t=t, b=b):
                process(t, b, True)

            @pl.when(t + 2 >= nw)
            def _(t=t, b=b):
                process(t, b, False)
        return 0

    lax.fori_loop(0, nw // 2, pair_body, 0)
    process(nw - 1, (nw - 1) % 2, False)

    pltpu.sync_copy(nrm_v, norms_hbm.at[:, wid])
    plsc.subcore_barrier()
    _readout_acc_slice(acc, sd_part_hbm, c, s)


def _k3_body(sd_hbm, src_hbm, dst_hbm, score_hbm,
             hdp_hbm,
             idx_s0, idx_d0, idx_s1, idx_d1,
             sc0, sc1, buf0, buf1, acc,
             sem0, sem1):
    c = lax.axis_index("c")
    s = lax.axis_index("s")
    wid = s * NC + c
    idx_s = (idx_s0, idx_s1)
    idx_d = (idx_d0, idx_d1)
    sc = (sc0, sc1)
    buf = (buf0, buf1)
    sem = (sem0, sem1)

    _zero_acc_slice(buf0, acc, s)
    plsc.subcore_barrier()

    nw = (NCHUNKS - wid + NW - 1) // NW

    def start_gather(t, b):
        base = (wid + t * NW) * CK
        pltpu.sync_copy(src_hbm.at[pl.ds(base, CK)], idx_s[b])
        pltpu.sync_copy(dst_hbm.at[pl.ds(base, CK)], idx_d[b])
        pltpu.sync_copy(score_hbm.at[pl.ds(base, CK)], sc[b])
        pltpu.async_copy(sd_hbm.at[idx_s[b]], buf[b], sem[b])

    for b in range(2):
        @pl.when(b < nw)
        def _(b=b):
            start_gather(b, b)

    def process(t, b):
        pltpu.make_async_copy(sd_hbm.at[idx_s[b]], buf[b], sem[b]).wait()

        def row_body(r, _rc):
            grp = sc[b][pl.ds((r // 16) * 16, 16)]
            lane = jnp.full((16,), r % 16, jnp.int32)
            sval = lax.gather(
                grp, lane[:, None],
                lax.GatherDimensionNumbers(offset_dims=(),
                                           collapsed_slice_dims=(0,),
                                           start_index_map=(0,)),
                (1,), mode=lax.GatherScatterMode.PROMISE_IN_BOUNDS)
            for j in range(D // 16):
                ds = pl.ds(j * 16, 16)
                buf[b][r, ds] = buf[b][r, ds] * sval
            return 0

        lax.fori_loop(0, CK, row_body, 0)
        pltpu.sync_copy(buf[b], acc.at[idx_d[b]], add=True)

        @pl.when(t + 2 < nw)
        def _():
            start_gather(t + 2, b)

    def pair_body(i2, _):
        for b in range(2):
            t = i2 * 2 + b

            @pl.when(t < nw)
            def _(t=t, b=b):
                process(t, b)
        return 0

    lax.fori_loop(0, (nw + 1) // 2, pair_body, 0)

    plsc.subcore_barrier()
    _readout_acc_slice(acc, hdp_hbm, c, s)


def _score_body(norms_ref, prod_ref, p_ref, out_ref):
    nsq = jnp.sum(norms_ref[...], axis=1)
    scale = jnp.sqrt(nsq[0]) * jnp.sqrt(nsq[1]) + 1e-06
    x = prod_ref[...]
    xh = x.astype(jnp.bfloat16)
    xl = (x - xh.astype(jnp.float32)).astype(jnp.bfloat16)
    p = p_ref[...]
    ph = p.astype(jnp.bfloat16)
    pl_ = (p - ph.astype(jnp.float32)).astype(jnp.bfloat16)
    dn = (((0,), (1,)), ((), ()))
    t = (lax.dot_general(ph, xh, dn, preferred_element_type=jnp.float32)
         + lax.dot_general(pl_, xh, dn, preferred_element_type=jnp.float32)
         + lax.dot_general(ph, xl, dn, preferred_element_type=jnp.float32))
    t = jax.nn.relu(t)
    out_ref[...] = jnp.exp(jnp.clip(jnp.sum(t, axis=0) / scale, -5.0, 5.0))


def _combine_body(a_ref, out_ref):
    out_ref[...] = a_ref[0] + a_ref[1]


def _ffn_body(hp_ref, w_ref, b_ref, out_ref):
    x = hp_ref[0] + hp_ref[1]
    y = lax.dot_general(x, w_ref[...], (((1,), (1,)), ((), ())),
                        preferred_element_type=jnp.float32)
    out_ref[...] = jax.nn.relu(y + b_ref[...])


_sc_mesh = plsc.VectorSubcoreMesh(core_axis_name="c", subcore_axis_name="s")

_k1 = pl.kernel(
    _k1_body,
    out_type=[
        jax.ShapeDtypeStruct((E_PAD, D), jnp.float32),
        jax.ShapeDtypeStruct((NC, N_PAD, D), jnp.float32),
        jax.ShapeDtypeStruct((2, NW, 16), jnp.float32),
    ],
    mesh=_sc_mesh,
    scratch_types=[
        pltpu.VMEM((CK1,), jnp.int32),
        pltpu.VMEM((CK1,), jnp.int32),
        pltpu.VMEM((CK1,), jnp.int32),
        pltpu.VMEM((CK1,), jnp.int32),
        pltpu.VMEM((CK1, D), jnp.float32),
        pltpu.VMEM((CK1, D), jnp.float32),
        pltpu.VMEM((CK1, D), jnp.float32),
        pltpu.VMEM((CK1, D), jnp.float32),
        pltpu.VMEM((2, 16), jnp.float32),
        pltpu.VMEM_SHARED((N_PAD, D), jnp.float32),
        pltpu.SemaphoreType.DMA,
        pltpu.SemaphoreType.DMA,
        pltpu.SemaphoreType.DMA,
        pltpu.SemaphoreType.DMA,
    ],
    name="k1_gather_prod_segsum",
)

_k3 = pl.kernel(
    _k3_body,
    out_type=jax.ShapeDtypeStruct((NC, N_PAD, D), jnp.float32),
    mesh=_sc_mesh,
    scratch_types=[
        pltpu.VMEM((CK,), jnp.int32),
        pltpu.VMEM((CK,), jnp.int32),
        pltpu.VMEM((CK,), jnp.int32),
        pltpu.VMEM((CK,), jnp.int32),
        pltpu.VMEM((CK,), jnp.float32),
        pltpu.VMEM((CK,), jnp.float32),
        pltpu.VMEM((CK, D), jnp.float32),
        pltpu.VMEM((CK, D), jnp.float32),
        pltpu.VMEM_SHARED((N_PAD, D), jnp.float32),
        pltpu.SemaphoreType.DMA,
        pltpu.SemaphoreType.DMA,
    ],
    name="k3_weighted_segsum",
)


def kernel(h, edge_index, proj_cosim, ffn_w, ffn_b):
    src = edge_index[0]
    dst = edge_index[1]

    prod, sd_part, norms = _k1(h, src, dst)

    score = pl.pallas_call(
        _score_body,
        grid=(E_PAD // BE,),
        in_specs=[
            pl.BlockSpec((2, NW * 16), lambda i: (0, 0)),
            pl.BlockSpec((BE, D), lambda i: (i, 0)),
            pl.BlockSpec((D, D), lambda i: (0, 0)),
        ],
        out_specs=pl.BlockSpec((BE,), lambda i: (i,)),
        out_shape=jax.ShapeDtypeStruct((E_PAD,), jnp.float32),
    )(norms.reshape(2, NW * 16), prod, proj_cosim)

    sd = pl.pallas_call(
        _combine_body,
        grid=(4,),
        in_specs=[pl.BlockSpec((NC, 2528, D), lambda i: (0, i, 0))],
        out_specs=pl.BlockSpec((2528, D), lambda i: (i, 0)),
        out_shape=jax.ShapeDtypeStruct((N_PAD, D), jnp.float32),
    )(sd_part)

    hd_part = _k3(sd, src, dst, score)

    out = pl.pallas_call(
        _ffn_body,
        grid=(N // BN,),
        in_specs=[
            pl.BlockSpec((NC, BN, D), lambda i: (0, i, 0)),
            pl.BlockSpec((D, D), lambda i: (0, 0)),
            pl.BlockSpec((1, D), lambda i: (0, 0)),
        ],
        out_specs=pl.BlockSpec((BN, D), lambda i: (i, 0)),
        out_shape=jax.ShapeDtypeStruct((N, D), jnp.float32),
    )(hd_part, ffn_w, ffn_b.reshape(1, D))

    return out
